# Initial kernel scaffold; baseline (speedup 1.0000x reference)
import jax
import jax.numpy as jnp
from jax.experimental import pallas as pl


def kernel(x, edge_index, edge_weight, W_v2e1, b_v2e1, W_e2v1, b_e2v1, W_v2e2, b_v2e2, W_e2v2, b_e2v2):
    raise NotImplementedError("write your pallas kernel here")


# speedup vs baseline: 4.1135x; 4.1135x over previous
import jax
import jax.numpy as jnp
from jax import lax
from jax.experimental import pallas as pl
from jax.experimental.pallas import tpu as pltpu
from jax.experimental.pallas import tpu_sc as plsc

N = 10000
H = 5000
E = 320000
D = 128

NC, NS = 2, 16
NW = NC * NS
C = 128
NCHUNK = 79
EPT = NCHUNK * C
E_PAD = NW * EPT
NDST_E = 5120
RPT_E = NDST_E // NS
NDST_V = 10112
RPT_V = NDST_V // NS

_MESH = plsc.VectorSubcoreMesh(core_axis_name="c", subcore_axis_name="s")


def _make_seg_sum(n_dst, rpt):
    def body(table, src_idx, dst_idx, zeros, out, src_v, dst_v, rows_v, accum, sem):
        c = lax.axis_index("c")
        s = lax.axis_index("s")
        wid = s * NC + c
        pltpu.sync_copy(src_idx.at[wid], src_v)
        pltpu.sync_copy(dst_idx.at[wid], dst_v)
        pltpu.sync_copy(zeros.at[pl.ds(0, rpt)], accum.at[pl.ds(s * rpt, rpt)])
        plsc.subcore_barrier()

        def chunk(j, carry):
            pltpu.async_copy(table.at[src_v.at[j]], rows_v, sem).wait()
            pltpu.sync_copy(rows_v, accum.at[dst_v.at[j]], add=True)
            return carry

        lax.fori_loop(0, NCHUNK, chunk, 0)
        plsc.subcore_barrier()
        pltpu.sync_copy(accum.at[pl.ds(s * rpt, rpt)],
                        out.at[c, pl.ds(s * rpt, rpt)])

    return pl.kernel(
        body,
        out_type=jax.ShapeDtypeStruct((NC, n_dst, D), jnp.float32),
        mesh=_MESH,
        scratch_types=[
            pltpu.VMEM((NCHUNK, C), jnp.int32),
            pltpu.VMEM((NCHUNK, C), jnp.int32),
            pltpu.VMEM((C, D), jnp.float32),
            pltpu.VMEM_SHARED((n_dst, D), jnp.float32),
            pltpu.SemaphoreType.DMA,
        ],
    )


_seg_e = _make_seg_sum(NDST_E, RPT_E)
_seg_v = _make_seg_sum(NDST_V, RPT_V)


def _make_ones_scatter(n_dst, rpt):
    def body(dst_idx, ones, zeros, out, dst_v, ones_v, accum):
        c = lax.axis_index("c")
        s = lax.axis_index("s")
        wid = s * NC + c
        pltpu.sync_copy(dst_idx.at[wid], dst_v)
        pltpu.sync_copy(ones, ones_v)
        pltpu.sync_copy(zeros.at[pl.ds(0, rpt)], accum.at[pl.ds(s * rpt, rpt)])
        plsc.subcore_barrier()

        def chunk(j, carry):
            pltpu.sync_copy(ones_v, accum.at[dst_v.at[j]], add=True)
            return carry

        lax.fori_loop(0, NCHUNK, chunk, 0)
        plsc.subcore_barrier()
        pltpu.sync_copy(accum.at[pl.ds(s * rpt, rpt)],
                        out.at[c, pl.ds(s * rpt, rpt)])

    return pl.kernel(
        body,
        out_type=jax.ShapeDtypeStruct((NC, n_dst, D), jnp.float32),
        mesh=_MESH,
        scratch_types=[
            pltpu.VMEM((NCHUNK, C), jnp.int32),
            pltpu.VMEM((C, D), jnp.float32),
            pltpu.VMEM_SHARED((n_dst, D), jnp.float32),
        ],
    )


_deg_e = _make_ones_scatter(NDST_E, RPT_E)
_deg_v = _make_ones_scatter(NDST_V, RPT_V)


def _mm_body(s_ref, d_ref, wv_ref, bv_ref, we_ref, o_ref):
    s = s_ref[0] + s_ref[1]
    deg = jnp.maximum(d_ref[0, :, 0:1] + d_ref[1, :, 0:1], 1.0)
    xe = jnp.dot(s / deg, wv_ref[...], preferred_element_type=jnp.float32)
    xe = jnp.maximum(xe + bv_ref[...], 0.0)
    o_ref[...] = jnp.dot(xe, we_ref[...], preferred_element_type=jnp.float32)


_mm = pl.pallas_call(
    _mm_body, out_shape=jax.ShapeDtypeStruct((NDST_E, D), jnp.float32))


def _node_relu_body(s_ref, d_ref, b_ref, o_ref):
    s = s_ref[0] + s_ref[1]
    deg = jnp.maximum(d_ref[0, :, 0:1] + d_ref[1, :, 0:1], 1.0)
    o_ref[...] = jnp.maximum(s / deg + b_ref[...], 0.0)[:N]


_node_relu = pl.pallas_call(
    _node_relu_body, out_shape=jax.ShapeDtypeStruct((N, D), jnp.float32))


def _node_body(s_ref, d_ref, b_ref, o_ref):
    s = s_ref[0] + s_ref[1]
    deg = jnp.maximum(d_ref[0, :, 0:1] + d_ref[1, :, 0:1], 1.0)
    o_ref[...] = (s / deg + b_ref[...])[:N]


_node = pl.pallas_call(
    _node_body, out_shape=jax.ShapeDtypeStruct((N, D), jnp.float32))


def kernel(x, edge_index, edge_weight,
           W_v2e1, b_v2e1, W_e2v1, b_e2v1,
           W_v2e2, b_v2e2, W_e2v2, b_e2v2):
    del edge_weight
    v = edge_index[0].astype(jnp.int32)
    e = edge_index[1].astype(jnp.int32)
    pad = E_PAD - E
    vsrc = jnp.concatenate([v, jnp.zeros((pad,), jnp.int32)]).reshape(NW, NCHUNK, C)
    esrc = jnp.concatenate([e, jnp.zeros((pad,), jnp.int32)]).reshape(NW, NCHUNK, C)
    vdst = jnp.concatenate([v, jnp.full((pad,), N, jnp.int32)]).reshape(NW, NCHUNK, C)
    edst = jnp.concatenate([e, jnp.full((pad,), H, jnp.int32)]).reshape(NW, NCHUNK, C)
    zeros_f = jnp.zeros((RPT_V, D), jnp.float32)
    ones_f = jnp.ones((C, D), jnp.float32)

    dege_p = _deg_e(edst, ones_f, zeros_f)
    degv_p = _deg_v(vdst, ones_f, zeros_f)
    bv1 = b_v2e1.reshape(1, D)
    be1 = b_e2v1.reshape(1, D)
    bv2 = b_v2e2.reshape(1, D)
    be2 = b_e2v2.reshape(1, D)

    s1 = _seg_e(x, vsrc, edst, zeros_f)
    t1 = _mm(s1, dege_p, W_v2e1, bv1, W_e2v1)
    s2 = _seg_v(t1, esrc, vdst, zeros_f)
    h = _node_relu(s2, degv_p, be1)
    s3 = _seg_e(h, vsrc, edst, zeros_f)
    t2 = _mm(s3, dege_p, W_v2e2, bv2, W_e2v2)
    s4 = _seg_v(t2, esrc, vdst, zeros_f)
    return _node(s4, degv_p, be2)
